# Initial kernel scaffold; baseline (speedup 1.0000x reference)
import functools

import jax
import jax.numpy as jnp
from jax import lax
from jax.experimental import pallas as pl
from jax.experimental.pallas import tpu as pltpu
from jax.experimental.pallas import tpu_sc as plsc

N = 10000
D = 128
T = 7
NSEG = N * T
NSEGP = 71680
E = 320000
EPAD = 327680
NW = 32
EW = EPAD // NW
ETILE = EPAD // 16
B = 128
NPAD = 10240

_mesh = plsc.VectorSubcoreMesh(core_axis_name="c", subcore_axis_name="s")


def _iota16():
    return lax.broadcasted_iota(jnp.int32, (16,), 0)


def _mm_body(x_ref, w_ref, u_ref):
    u_ref[0] = jnp.dot(x_ref[...], w_ref[0], preferred_element_type=jnp.float32)


def _compute_u(x, w3):
    return pl.pallas_call(
        _mm_body,
        grid=(T, 5),
        in_specs=[
            pl.BlockSpec((N // 5, D), lambda t, j: (j, 0)),
            pl.BlockSpec((1, D, D), lambda t, j: (t, 0, 0)),
        ],
        out_specs=pl.BlockSpec((1, N // 5, D), lambda t, j: (t, j, 0)),
        out_shape=jax.ShapeDtypeStruct((T, N, D), jnp.float32),
    )(x, w3)


@functools.partial(
    pl.kernel,
    out_type=jax.ShapeDtypeStruct((NSEGP,), jnp.float32),
    mesh=_mesh,
    scratch_types=[
        pltpu.VMEM_SHARED((NSEGP, 16), jnp.float32),
        pltpu.VMEM((B, 16), jnp.float32),
        pltpu.VMEM((B, 16), jnp.float32),
        pltpu.VMEM((B,), jnp.int32),
        pltpu.VMEM((B,), jnp.int32),
        pltpu.VMEM((B,), jnp.int32),
        pltpu.VMEM((B, 16), jnp.float32),
        pltpu.VMEM((B,), jnp.float32),
    ],
)
def _sc_counts(rowp, typp, inv_out, counts_sp, ones16, z16, rowb, typb, segb,
               cchunk, dinv):
    cid = lax.axis_index("c")
    sid = lax.axis_index("s")
    iota = _iota16()
    zero16i = jnp.zeros((16,), jnp.int32)

    @pl.when(cid == 0)
    def _():
        one_row = jnp.where(iota == 0, 1.0, 0.0).astype(jnp.float32)
        zrow = jnp.zeros((16,), jnp.float32)

        def fill(r, _):
            ones16[r, :] = one_row
            z16[r, :] = zrow
            return 0
        lax.fori_loop(0, B, fill, 0)

        def zslice(q, _):
            pltpu.sync_copy(z16, counts_sp.at[pl.ds(sid * 4480 + q * B, B)])
            return 0
        lax.fori_loop(0, 35, zslice, 0)
        plsc.subcore_barrier()

        def count_batch(b, _):
            base = sid * ETILE + b * B
            pltpu.sync_copy(rowp.at[pl.ds(base, B)], rowb)
            pltpu.sync_copy(typp.at[pl.ds(base, B)], typb)
            for j in range(8):
                sl = pl.ds(j * 16, 16)
                segb[sl] = rowb[sl] * 7 + typb[sl]
            pltpu.sync_copy(ones16, counts_sp.at[segb], add=True)
            return 0
        lax.fori_loop(0, ETILE // B, count_batch, 0)
        plsc.subcore_barrier()

        def inv_chunk(q, _):
            r0 = sid * 4480 + q * B
            pltpu.sync_copy(counts_sp.at[pl.ds(r0, B)], cchunk)
            for j in range(8):
                ji = j * 16 + iota
                c16 = plsc.load_gather(cchunk, [ji, zero16i])
                dinv[pl.ds(j * 16, 16)] = jnp.where(c16 > 0.0, 1.0 / c16, 0.0)
            pltpu.sync_copy(dinv, inv_out.at[pl.ds(r0, B)])
            return 0
        lax.fori_loop(0, 35, inv_chunk, 0)


@functools.partial(
    pl.kernel,
    out_type=jax.ShapeDtypeStruct((2, NPAD, D), jnp.float32),
    mesh=_mesh,
    scratch_types=[
        pltpu.VMEM_SHARED((NPAD, D), jnp.float32),
        pltpu.VMEM((NSEGP,), jnp.float32),
        pltpu.VMEM((B,), jnp.int32),
        pltpu.VMEM((B,), jnp.int32),
        pltpu.VMEM((B,), jnp.int32),
        pltpu.VMEM((B,), jnp.int32),
        pltpu.VMEM((B,), jnp.float32),
        pltpu.VMEM((B, D), jnp.float32),
        pltpu.SemaphoreType.DMA,
    ],
)
def _sc_main(rowp, colp, typp, u2d, invh, out, acc_sp, invtab, rowb, colb,
             typb, gixb, invb, ubuf, sem):
    cid = lax.axis_index("c")
    sid = lax.axis_index("s")
    wid = cid * 16 + sid
    zrow = jnp.zeros((16,), jnp.float32)

    def zub(r, _):
        for k in range(8):
            ubuf[r, pl.ds(k * 16, 16)] = zrow
        return 0
    lax.fori_loop(0, B, zub, 0)

    def zacc(q, _):
        pltpu.sync_copy(ubuf, acc_sp.at[pl.ds(sid * 640 + q * B, B)])
        return 0
    lax.fori_loop(0, 5, zacc, 0)

    pltpu.sync_copy(invh, invtab)
    plsc.subcore_barrier()

    def batch(b, _):
        base = wid * EW + b * B
        pltpu.sync_copy(rowp.at[pl.ds(base, B)], rowb)
        pltpu.sync_copy(colp.at[pl.ds(base, B)], colb)
        pltpu.sync_copy(typp.at[pl.ds(base, B)], typb)
        for j in range(8):
            sl = pl.ds(j * 16, 16)
            t16 = typb[sl]
            invb[sl] = plsc.load_gather(invtab, [rowb[sl] * 7 + t16])
            gixb[sl] = t16 * N + colb[sl]
        pltpu.async_copy(u2d.at[gixb], ubuf, sem).wait()

        def scale(e, _):
            s = invb[e]
            for k in range(8):
                sl = pl.ds(k * 16, 16)
                ubuf[e, sl] = ubuf[e, sl] * s
            return 0
        lax.fori_loop(0, B, scale, 0)

        pltpu.sync_copy(ubuf, acc_sp.at[rowb], add=True)
        return 0
    lax.fori_loop(0, EW // B, batch, 0)
    plsc.subcore_barrier()

    def wout(q, _):
        r0 = sid * 640 + q * B
        pltpu.sync_copy(acc_sp.at[pl.ds(r0, B)], out.at[cid, pl.ds(r0, B)])
        return 0
    lax.fori_loop(0, 5, wout, 0)


def _add_body(p_ref, o_ref):
    o_ref[...] = p_ref[0] + p_ref[1]


def _combine(p):
    return pl.pallas_call(
        _add_body,
        grid=(5,),
        in_specs=[pl.BlockSpec((2, N // 5, D), lambda j: (0, j, 0))],
        out_specs=pl.BlockSpec((N // 5, D), lambda j: (j, 0)),
        out_shape=jax.ShapeDtypeStruct((N, D), jnp.float32),
    )(p)


def kernel(x, edge_index, edge_type, weights):
    row = edge_index[0].astype(jnp.int32)
    col = edge_index[1].astype(jnp.int32)
    typ = edge_type.astype(jnp.int32)
    pad = EPAD - E
    rowp = jnp.concatenate([row, jnp.full((pad,), N, jnp.int32)])
    colp = jnp.concatenate([col, jnp.zeros((pad,), jnp.int32)])
    typp = jnp.concatenate([typ, jnp.zeros((pad,), jnp.int32)])
    w3 = weights.reshape(T, D, D)

    u2d = _compute_u(x, w3).reshape(NSEG, D)
    inv = _sc_counts(rowp, typp)
    part = _sc_main(rowp, colp, typp, u2d, inv)
    return _combine(part[:, :N, :])


# speedup vs baseline: 2.6956x; 2.6956x over previous
import functools

import jax
import jax.numpy as jnp
from jax import lax
from jax.experimental import pallas as pl
from jax.experimental.pallas import tpu as pltpu
from jax.experimental.pallas import tpu_sc as plsc

N = 10000
D = 128
T = 7
NSEG = N * T
NSEGP = 71680
E = 320000
EPAD = 327680
NW = 32
EW = EPAD // NW
ETILE = EPAD // 16
B = 128
NPAD = 10240

_mesh = plsc.VectorSubcoreMesh(core_axis_name="c", subcore_axis_name="s")


def _iota16():
    return lax.broadcasted_iota(jnp.int32, (16,), 0)


def _mm_body(x_ref, w_ref, u_ref):
    u_ref[0] = jnp.dot(x_ref[...], w_ref[0], preferred_element_type=jnp.float32)


def _compute_u(x, w3):
    return pl.pallas_call(
        _mm_body,
        grid=(T, 5),
        in_specs=[
            pl.BlockSpec((N // 5, D), lambda t, j: (j, 0)),
            pl.BlockSpec((1, D, D), lambda t, j: (t, 0, 0)),
        ],
        out_specs=pl.BlockSpec((1, N // 5, D), lambda t, j: (t, j, 0)),
        out_shape=jax.ShapeDtypeStruct((T, N, D), jnp.float32),
    )(x, w3)


@functools.partial(
    pl.kernel,
    out_type=jax.ShapeDtypeStruct((NSEGP,), jnp.float32),
    mesh=_mesh,
    compiler_params=pltpu.CompilerParams(needs_layout_passes=False),
    scratch_types=[
        pltpu.VMEM_SHARED((NSEGP,), jnp.float32),
        pltpu.VMEM((B,), jnp.float32),
        pltpu.VMEM((B,), jnp.int32),
        pltpu.VMEM((B,), jnp.int32),
        pltpu.VMEM((B,), jnp.int32),
        pltpu.VMEM((2240,), jnp.float32),
        pltpu.VMEM((2240,), jnp.float32),
    ],
)
def _sc_counts(rowp, typp, inv_out, counts_sp, ones, rowb, typb, segb,
               cchunk, dinv):
    cid = lax.axis_index("c")
    sid = lax.axis_index("s")

    @pl.when(cid == 0)
    def _():
        one_v = jnp.full((16,), 1.0, jnp.float32)
        zero_v = jnp.zeros((16,), jnp.float32)

        def fill(r, _):
            ones[pl.ds(r * 16, 16)] = one_v
            dinv[pl.ds(r * 16, 16)] = zero_v
            return 0
        lax.fori_loop(0, 140, fill, 0)

        def zslice(q, _):
            pltpu.sync_copy(dinv, counts_sp.at[pl.ds(sid * 4480 + q * 2240, 2240)])
            return 0
        lax.fori_loop(0, 2, zslice, 0)
        plsc.subcore_barrier()

        def count_batch(b, _):
            base = sid * ETILE + b * B
            pltpu.sync_copy(rowp.at[pl.ds(base, B)], rowb)
            pltpu.sync_copy(typp.at[pl.ds(base, B)], typb)
            for j in range(8):
                sl = pl.ds(j * 16, 16)
                segb[sl] = rowb[sl] * 7 + typb[sl]
            pltpu.sync_copy(ones, counts_sp.at[segb], add=True)
            return 0
        lax.fori_loop(0, ETILE // B, count_batch, 0)
        plsc.subcore_barrier()

        def inv_chunk(q, _):
            r0 = sid * 4480 + q * 2240
            pltpu.sync_copy(counts_sp.at[pl.ds(r0, 2240)], cchunk)

            def inv_row(r, _):
                sl = pl.ds(r * 16, 16)
                c16 = cchunk[sl]
                dinv[sl] = jnp.where(c16 > 0.0, 1.0 / c16, 0.0)
                return 0
            lax.fori_loop(0, 140, inv_row, 0)
            pltpu.sync_copy(dinv, inv_out.at[pl.ds(r0, 2240)])
            return 0
        lax.fori_loop(0, 2, inv_chunk, 0)


@functools.partial(
    pl.kernel,
    out_type=jax.ShapeDtypeStruct((2, NPAD, D), jnp.float32),
    mesh=_mesh,
    compiler_params=pltpu.CompilerParams(needs_layout_passes=False),
    scratch_types=[
        pltpu.VMEM_SHARED((NPAD, D), jnp.float32),
        pltpu.VMEM((B,), jnp.int32),
        pltpu.VMEM((B,), jnp.int32),
        pltpu.VMEM((B,), jnp.int32),
        pltpu.VMEM((B,), jnp.int32),
        pltpu.VMEM((B,), jnp.int32),
        pltpu.VMEM((B,), jnp.float32),
        pltpu.VMEM((B, D), jnp.float32),
        pltpu.SemaphoreType.DMA,
        pltpu.SemaphoreType.DMA,
    ],
)
def _sc_main(rowp, colp, typp, u2d, invh, out, acc_sp, rowb, colb,
             typb, segb, gixb, invb, ubuf, sem, sem2):
    cid = lax.axis_index("c")
    sid = lax.axis_index("s")
    wid = cid * 16 + sid
    zrow = jnp.zeros((16,), jnp.float32)

    def zub(r, _):
        for k in range(8):
            ubuf[r, pl.ds(k * 16, 16)] = zrow
        return 0
    lax.fori_loop(0, B, zub, 0)

    def zacc(q, _):
        pltpu.sync_copy(ubuf, acc_sp.at[pl.ds(sid * 640 + q * B, B)])
        return 0
    lax.fori_loop(0, 5, zacc, 0)

    plsc.subcore_barrier()

    def batch(b, _):
        base = wid * EW + b * B
        pltpu.sync_copy(rowp.at[pl.ds(base, B)], rowb)
        pltpu.sync_copy(colp.at[pl.ds(base, B)], colb)
        pltpu.sync_copy(typp.at[pl.ds(base, B)], typb)
        for j in range(8):
            sl = pl.ds(j * 16, 16)
            t16 = typb[sl]
            segb[sl] = rowb[sl] * 7 + t16
            gixb[sl] = t16 * N + colb[sl]
        d1 = pltpu.async_copy(u2d.at[gixb], ubuf, sem)
        d2 = pltpu.async_copy(invh.at[segb], invb, sem2)
        d1.wait()
        d2.wait()

        def scale(e, _):
            sv = plsc.load_gather(invb, [jnp.full((16,), e, jnp.int32)])
            for k in range(8):
                sl = pl.ds(k * 16, 16)
                ubuf[e, sl] = ubuf[e, sl] * sv
            return 0
        lax.fori_loop(0, B, scale, 0)

        pltpu.sync_copy(ubuf, acc_sp.at[rowb], add=True)
        return 0
    lax.fori_loop(0, EW // B, batch, 0)
    plsc.subcore_barrier()

    def wout(q, _):
        r0 = sid * 640 + q * B
        pltpu.sync_copy(acc_sp.at[pl.ds(r0, B)], out.at[cid, pl.ds(r0, B)])
        return 0
    lax.fori_loop(0, 5, wout, 0)


def _add_body(p_ref, o_ref):
    o_ref[...] = p_ref[0] + p_ref[1]


def _combine(p):
    return pl.pallas_call(
        _add_body,
        grid=(5,),
        in_specs=[pl.BlockSpec((2, N // 5, D), lambda j: (0, j, 0))],
        out_specs=pl.BlockSpec((N // 5, D), lambda j: (j, 0)),
        out_shape=jax.ShapeDtypeStruct((N, D), jnp.float32),
    )(p)


def kernel(x, edge_index, edge_type, weights):
    row = edge_index[0].astype(jnp.int32)
    col = edge_index[1].astype(jnp.int32)
    typ = edge_type.astype(jnp.int32)
    pad = EPAD - E
    rowp = jnp.concatenate([row, jnp.full((pad,), N, jnp.int32)])
    colp = jnp.concatenate([col, jnp.zeros((pad,), jnp.int32)])
    typp = jnp.concatenate([typ, jnp.zeros((pad,), jnp.int32)])
    w3 = weights.reshape(T, D, D)

    u2d = _compute_u(x, w3).reshape(NSEG, D)
    inv = _sc_counts(rowp, typp)
    part = _sc_main(rowp, colp, typp, u2d, inv)
    return _combine(part[:, :N, :])
